# baseline (speedup 1.0000x reference)
import functools

import jax
import jax.numpy as jnp
from jax import lax
from jax.experimental import pallas as pl
from jax.experimental.pallas import tpu as pltpu
from jax.experimental.pallas import tpu_sc as plsc

NUM_Q = 8
CODEBOOK_SIZE = 1024
CODE_DIM = 64
B, N = 16, 576

NC, NS, L = 2, 16, 16
NW = NC * NS
T = B * N
TPW = T // NW
CHUNK_T = 16
CHUNK_R = CHUNK_T * NUM_Q
NCHUNK = TPW // CHUNK_T
IDX_ROWS = TPW * NUM_Q // 128


ROWP = 65


def _body(idx_hbm, cb_hbm, out_hbm, idx_v, rows0, rows1, acc, accT, sem0, sem1, osem):
    wid = lax.axis_index("c") * NS + lax.axis_index("s")
    b = wid // 2
    n0 = (wid % 2) * TPW

    with jax.named_scope("ph_idx"):
        pltpu.sync_copy(idx_hbm.at[pl.ds(wid * IDX_ROWS, IDX_ROWS)], idx_v)

        iota = lax.iota(jnp.int32, L)
        qpat = (iota % NUM_Q) * CODEBOOK_SIZE

        for c in range(IDX_ROWS):
            for k in range(128 // L):
                sl = pl.ds(k * L, L)
                idx_v[c, sl] = idx_v[c, sl] + qpat

    rows = (rows0, rows1)
    sems = (sem0, sem1)

    def gather(c, buf):
        return pltpu.make_async_copy(cb_hbm.at[idx_v.at[c]], rows[buf], sems[buf])

    gather(0, 0).start()
    gather(1, 1).start()

    def compute(c, buf):
        @pl.when(c < 2)
        def _w():
            gather(c, buf).wait()
        chunk_base = c * CHUNK_T * ROWP
        for j in range(CHUNK_T):
            base = j * NUM_Q
            for r in range(CODE_DIM // L):
                sl = pl.ds(r * L, L)
                s = rows[buf][base, sl]
                for q in range(1, NUM_Q):
                    s = s + rows[buf][base + q, sl]
                acc[pl.ds(chunk_base + j * ROWP + r * L, L)] = s

    def chunk_pair(i, _):
        c0 = i * 2
        for buf in range(2):
            c = c0 + buf
            compute(c, buf)

            @pl.when(c + 2 < 2)
            def _start():
                gather(c + 2, buf).start()

        return _

    with jax.named_scope("ph_main"):
        lax.fori_loop(0, NCHUNK // 2, chunk_pair, 0, unroll=False)

    colbase = iota * ROWP

    def tpass(g, _):
        rowb = g * L
        for d in range(CODE_DIM):
            v = plsc.load_gather(acc, [colbase + (rowb * ROWP + d)])
            accT[pl.ds(d * TPW + rowb, L)] = v
        return _

    with jax.named_scope("ph_tpose"):
        lax.fori_loop(0, TPW // L, tpass, 0, unroll=False)

    obase = b * (CODE_DIM * N) + n0
    copies = [
        pltpu.make_async_copy(
            accT.at[pl.ds(d * TPW, TPW)],
            out_hbm.at[pl.ds(obase + d * N, TPW)],
            osem,
        )
        for d in range(CODE_DIM)
    ]
    with jax.named_scope("ph_out"):
        for cp in copies:
            cp.start()
        for cp in copies:
            cp.wait()


@jax.jit
def _vq_lookup(idx2d, cb2d):
    mesh = plsc.VectorSubcoreMesh(
        core_axis_name="c", subcore_axis_name="s", num_cores=NC, num_subcores=NS
    )
    f = pl.kernel(
        _body,
        out_type=jax.ShapeDtypeStruct((B * CODE_DIM * N,), jnp.float32),
        mesh=mesh,
        compiler_params=pltpu.CompilerParams(
            use_tc_tiling_on_sc=False, needs_layout_passes=False
        ),
        scratch_types=[
            pltpu.VMEM((IDX_ROWS, 128), jnp.int32),
            pltpu.VMEM((CHUNK_R, CODE_DIM), jnp.float32),
            pltpu.VMEM((CHUNK_R, CODE_DIM), jnp.float32),
            pltpu.VMEM((TPW * ROWP,), jnp.float32),
            pltpu.VMEM((CODE_DIM * TPW,), jnp.float32),
            pltpu.SemaphoreType.DMA,
            pltpu.SemaphoreType.DMA,
            pltpu.SemaphoreType.DMA,
        ],
    )
    return f(idx2d, cb2d)


def kernel(indices, codebooks):
    idx2d = indices.astype(jnp.int32).reshape(T * NUM_Q // 128, 128)
    cb2d = codebooks.reshape(NUM_Q * CODEBOOK_SIZE, CODE_DIM)
    return _vq_lookup(idx2d, cb2d).reshape(B, CODE_DIM, N)


# speedup vs baseline: 1.0767x; 1.0767x over previous
import functools

import jax
import jax.numpy as jnp
from jax import lax
from jax.experimental import pallas as pl
from jax.experimental.pallas import tpu as pltpu
from jax.experimental.pallas import tpu_sc as plsc

NUM_Q = 8
CODEBOOK_SIZE = 1024
CODE_DIM = 64
B, N = 16, 576

NC, NS, L = 2, 16, 16
NW = NC * NS
T = B * N
TPW = T // NW
CHUNK_T = 16
CHUNK_R = CHUNK_T * NUM_Q
NCHUNK = TPW // CHUNK_T
IDX_ROWS = TPW * NUM_Q // 128


ROWP = 65


def _body(idx_hbm, cb_hbm, out_hbm, idx_v, rows0, rows1, acc, accT, sem0, sem1, osem):
    wid = lax.axis_index("c") * NS + lax.axis_index("s")
    b = wid // 2
    n0 = (wid % 2) * TPW

    with jax.named_scope("ph_idx"):
        pltpu.sync_copy(idx_hbm.at[pl.ds(wid * IDX_ROWS, IDX_ROWS)], idx_v)

        iota = lax.iota(jnp.int32, L)
        qpat = (iota % NUM_Q) * CODEBOOK_SIZE

        for c in range(IDX_ROWS):
            for k in range(128 // L):
                sl = pl.ds(k * L, L)
                idx_v[c, sl] = idx_v[c, sl] + qpat

    rows = (rows0, rows1)
    sems = (sem0, sem1)

    def gather(c, buf):
        return pltpu.make_async_copy(cb_hbm.at[idx_v.at[c]], rows[buf], sems[buf])

    gather(0, 0).start()
    gather(1, 1).start()

    def compute(c, buf):
        gather(c, buf).wait()
        chunk_base = c * CHUNK_T * ROWP
        for j in range(CHUNK_T):
            base = j * NUM_Q
            for r in range(CODE_DIM // L):
                sl = pl.ds(r * L, L)
                v = [rows[buf][base + q, sl] for q in range(NUM_Q)]
                v = [v[2 * k] + v[2 * k + 1] for k in range(4)]
                v = [v[0] + v[1], v[2] + v[3]]
                acc[pl.ds(chunk_base + j * ROWP + r * L, L)] = v[0] + v[1]

    def chunk_pair(i, _):
        c0 = i * 2
        for buf in range(2):
            c = c0 + buf
            compute(c, buf)

            @pl.when(c + 2 < NCHUNK)
            def _start():
                gather(c + 2, buf).start()

        return _

    with jax.named_scope("ph_main"):
        lax.fori_loop(0, NCHUNK // 2, chunk_pair, 0, unroll=False)

    colbase = iota * ROWP

    def tpass(g, _):
        rowb = g * L
        for d in range(CODE_DIM):
            v = plsc.load_gather(acc, [colbase + (rowb * ROWP + d)])
            accT[pl.ds(d * TPW + rowb, L)] = v
        return _

    with jax.named_scope("ph_tpose"):
        lax.fori_loop(0, TPW // L, tpass, 0, unroll=False)

    obase = b * (CODE_DIM * N) + n0
    copies = [
        pltpu.make_async_copy(
            accT.at[pl.ds(d * TPW, TPW)],
            out_hbm.at[pl.ds(obase + d * N, TPW)],
            osem,
        )
        for d in range(CODE_DIM)
    ]
    with jax.named_scope("ph_out"):
        for cp in copies:
            cp.start()
        for cp in copies:
            cp.wait()


@jax.jit
def _vq_lookup(idx2d, cb2d):
    mesh = plsc.VectorSubcoreMesh(
        core_axis_name="c", subcore_axis_name="s", num_cores=NC, num_subcores=NS
    )
    f = pl.kernel(
        _body,
        out_type=jax.ShapeDtypeStruct((B * CODE_DIM * N,), jnp.float32),
        mesh=mesh,
        compiler_params=pltpu.CompilerParams(
            use_tc_tiling_on_sc=False, needs_layout_passes=False
        ),
        scratch_types=[
            pltpu.VMEM((IDX_ROWS, 128), jnp.int32),
            pltpu.VMEM((CHUNK_R, CODE_DIM), jnp.float32),
            pltpu.VMEM((CHUNK_R, CODE_DIM), jnp.float32),
            pltpu.VMEM((TPW * ROWP,), jnp.float32),
            pltpu.VMEM((CODE_DIM * TPW,), jnp.float32),
            pltpu.SemaphoreType.DMA,
            pltpu.SemaphoreType.DMA,
            pltpu.SemaphoreType.DMA,
        ],
    )
    return f(idx2d, cb2d)


def kernel(indices, codebooks):
    idx2d = indices.astype(jnp.int32).reshape(T * NUM_Q // 128, 128)
    cb2d = codebooks.reshape(NUM_Q * CODEBOOK_SIZE, CODE_DIM)
    return _vq_lookup(idx2d, cb2d).reshape(B, CODE_DIM, N)


# speedup vs baseline: 1.3290x; 1.2344x over previous
import jax
import jax.numpy as jnp
from jax import lax
from jax.experimental import pallas as pl
from jax.experimental.pallas import tpu as pltpu
from jax.experimental.pallas import tpu_sc as plsc

NUM_Q = 8
CODEBOOK_SIZE = 1024
CODE_DIM = 64
B, N = 16, 576

NC, NS, L = 2, 16, 16
NW = NC * NS
T = B * N
TPW = T // NW
CHUNK_T = 48
NCHUNK = TPW // CHUNK_T
ROWP = 65


def _body(idx_hbm, cb_hbm, out_hbm, raw_v, idxq, acc, acc65, accT, gsem, osem):
    wid = lax.axis_index("c") * NS + lax.axis_index("s")
    b = wid // 2
    n0 = (wid % 2) * TPW

    iota = lax.iota(jnp.int32, L)

    with jax.named_scope("ph_idx"):
        pltpu.sync_copy(idx_hbm.at[pl.ds(wid * TPW * NUM_Q, TPW * NUM_Q)], raw_v)
        for q in range(NUM_Q):
            gat = iota * NUM_Q + q
            for g in range(TPW // L):
                v = plsc.load_gather(raw_v, [gat + g * L * NUM_Q])
                idxq[pl.ds(q * TPW + g * L, L)] = v + q * CODEBOOK_SIZE

    with jax.named_scope("ph_main"):
        zeros = jnp.zeros((L,), jnp.float32)

        def zrow(t, _):
            for r in range(CODE_DIM // L):
                acc[t, pl.ds(r * L, L)] = zeros
            return _

        lax.fori_loop(0, TPW, zrow, 0, unroll=4)

        copies = []
        for c in range(NCHUNK):
            dst = acc.at[pl.ds(c * CHUNK_T, CHUNK_T)]
            for q in range(NUM_Q):
                src = cb_hbm.at[idxq.at[pl.ds(q * TPW + c * CHUNK_T, CHUNK_T)]]
                copies.append(pltpu.make_async_copy(src, dst, gsem))
        for cp in copies:
            cp.start(add=True)
        for cp in copies:
            cp.wait()

    with jax.named_scope("ph_pitch"):
        def prow(t, _):
            for r in range(CODE_DIM // L):
                acc65[pl.ds(t * ROWP + r * L, L)] = acc[t, pl.ds(r * L, L)]
            return _

        lax.fori_loop(0, TPW, prow, 0, unroll=4)

    with jax.named_scope("ph_tpose"):
        colbase = iota * ROWP

        def tpass(g, _):
            rowb = g * L
            for d in range(CODE_DIM):
                v = plsc.load_gather(acc65, [colbase + (rowb * ROWP + d)])
                accT[pl.ds(d * TPW + rowb, L)] = v
            return _

        lax.fori_loop(0, TPW // L, tpass, 0, unroll=False)

    with jax.named_scope("ph_out"):
        obase = b * (CODE_DIM * N) + n0
        ocopies = [
            pltpu.make_async_copy(
                accT.at[pl.ds(d * TPW, TPW)],
                out_hbm.at[pl.ds(obase + d * N, TPW)],
                osem,
            )
            for d in range(CODE_DIM)
        ]
        for cp in ocopies:
            cp.start()
        for cp in ocopies:
            cp.wait()


@jax.jit
def _vq_lookup(idx1d, cb2d):
    mesh = plsc.VectorSubcoreMesh(
        core_axis_name="c", subcore_axis_name="s", num_cores=NC, num_subcores=NS
    )
    f = pl.kernel(
        _body,
        out_type=jax.ShapeDtypeStruct((B * CODE_DIM * N,), jnp.float32),
        mesh=mesh,
        compiler_params=pltpu.CompilerParams(
            use_tc_tiling_on_sc=False, needs_layout_passes=False
        ),
        scratch_types=[
            pltpu.VMEM((TPW * NUM_Q,), jnp.int32),
            pltpu.VMEM((TPW * NUM_Q,), jnp.int32),
            pltpu.VMEM((TPW, CODE_DIM), jnp.float32),
            pltpu.VMEM((TPW * ROWP,), jnp.float32),
            pltpu.VMEM((CODE_DIM * TPW,), jnp.float32),
            pltpu.SemaphoreType.DMA,
            pltpu.SemaphoreType.DMA,
        ],
    )
    return f(idx1d, cb2d)


def kernel(indices, codebooks):
    idx1d = indices.astype(jnp.int32).reshape(T * NUM_Q)
    cb2d = codebooks.reshape(NUM_Q * CODEBOOK_SIZE, CODE_DIM)
    return _vq_lookup(idx1d, cb2d).reshape(B, CODE_DIM, N)


# speedup vs baseline: 1.3469x; 1.0135x over previous
import jax
import jax.numpy as jnp
from jax import lax
from jax.experimental import pallas as pl
from jax.experimental.pallas import tpu as pltpu
from jax.experimental.pallas import tpu_sc as plsc

NUM_Q = 8
CODEBOOK_SIZE = 1024
CODE_DIM = 64
B, N = 16, 576

NC, NS, L = 2, 16, 16
NW = NC * NS
T = B * N
TPW = T // NW
CHUNK_T = 48
NCHUNK = TPW // CHUNK_T
ROWP = 65
GPC = TPW // L // NCHUNK


def _body(idx_hbm, cb_hbm, out_hbm, raw_v, idxq, acc, acc65, accT, *sems):
    gsems, osem = sems[:NCHUNK], sems[NCHUNK]
    wid = lax.axis_index("c") * NS + lax.axis_index("s")
    b = wid // 2
    n0 = (wid % 2) * TPW

    iota = lax.iota(jnp.int32, L)

    with jax.named_scope("ph_idx"):
        pltpu.sync_copy(idx_hbm.at[b, pl.ds(n0, TPW)], raw_v)
        for q in range(NUM_Q):
            cols = jnp.full((L,), q, jnp.int32)
            for g in range(TPW // L):
                v = plsc.load_gather(raw_v, [iota + g * L, cols])
                idxq[pl.ds(q * TPW + g * L, L)] = v + q * CODEBOOK_SIZE

    with jax.named_scope("ph_zero"):
        zeros = jnp.zeros((L,), jnp.float32)

        def zrow(t, _):
            for r in range(CODE_DIM // L):
                acc[t, pl.ds(r * L, L)] = zeros
            return _

        lax.fori_loop(0, TPW, zrow, 0, unroll=4)

    with jax.named_scope("ph_main"):
        copies = []
        for c in range(NCHUNK):
            dst = acc.at[pl.ds(c * CHUNK_T, CHUNK_T)]
            for q in range(NUM_Q):
                src = cb_hbm.at[idxq.at[pl.ds(q * TPW + c * CHUNK_T, CHUNK_T)]]
                copies.append(pltpu.make_async_copy(src, dst, gsems[c]))
        for cp in copies:
            cp.start(add=True)

    with jax.named_scope("ph_tpose"):
        def prow(t, _):
            for r in range(CODE_DIM // L):
                acc65[pl.ds(t * ROWP + r * L, L)] = acc[t, pl.ds(r * L, L)]
            return _

        colbase = iota * ROWP

        def tpass(g, _):
            rowb = g * L
            for d in range(CODE_DIM):
                v = plsc.load_gather(acc65, [colbase + (rowb * ROWP + d)])
                accT[pl.ds(d * TPW + rowb, L)] = v
            return _

        for c in range(NCHUNK):
            for cp in copies[c * NUM_Q : (c + 1) * NUM_Q]:
                cp.wait()
            lax.fori_loop(c * CHUNK_T, (c + 1) * CHUNK_T, prow, 0, unroll=4)
            lax.fori_loop(c * GPC, (c + 1) * GPC, tpass, 0, unroll=False)

    with jax.named_scope("ph_out"):
        ocopies = [
            pltpu.make_async_copy(
                accT.at[pl.ds(d * TPW, TPW)],
                out_hbm.at[b, d, pl.ds(n0, TPW)],
                osem,
            )
            for d in range(CODE_DIM)
        ]
        for cp in ocopies:
            cp.start()
        for cp in ocopies:
            cp.wait()


@jax.jit
def _vq_lookup(indices, cb2d):
    mesh = plsc.VectorSubcoreMesh(
        core_axis_name="c", subcore_axis_name="s", num_cores=NC, num_subcores=NS
    )
    f = pl.kernel(
        _body,
        out_type=jax.ShapeDtypeStruct((B, CODE_DIM, N), jnp.float32),
        mesh=mesh,
        compiler_params=pltpu.CompilerParams(
            use_tc_tiling_on_sc=False, needs_layout_passes=False
        ),
        scratch_types=[
            pltpu.VMEM((TPW, NUM_Q), jnp.int32),
            pltpu.VMEM((TPW * NUM_Q,), jnp.int32),
            pltpu.VMEM((TPW, CODE_DIM), jnp.float32),
            pltpu.VMEM((TPW * ROWP,), jnp.float32),
            pltpu.VMEM((CODE_DIM * TPW,), jnp.float32),
        ]
        + [pltpu.SemaphoreType.DMA] * (NCHUNK + 1),
    )
    return f(indices, cb2d)


def kernel(indices, codebooks):
    cb2d = codebooks.reshape(NUM_Q * CODEBOOK_SIZE, CODE_DIM)
    return _vq_lookup(indices.astype(jnp.int32), cb2d)


# speedup vs baseline: 1.3765x; 1.0219x over previous
import jax
import jax.numpy as jnp
from jax import lax
from jax.experimental import pallas as pl
from jax.experimental.pallas import tpu as pltpu
from jax.experimental.pallas import tpu_sc as plsc

NUM_Q = 8
CODEBOOK_SIZE = 1024
CODE_DIM = 64
B, N = 16, 576

NC, NS, L = 2, 16, 16
NW = NC * NS
T = B * N
TPW = T // NW
CHUNK_T = 48
NCHUNK = TPW // CHUNK_T
ROWP = 65
GPC = TPW // L // NCHUNK


def _body(idx_hbm, cb_hbm, out_hbm, raw_v, idxq, acc, acc65, accT, *sems):
    gsems, osem = sems[:NCHUNK], sems[NCHUNK]
    wid = lax.axis_index("c") * NS + lax.axis_index("s")
    b = wid // 2
    n0 = (wid % 2) * TPW

    iota = lax.iota(jnp.int32, L)

    with jax.named_scope("ph_idx"):
        pltpu.sync_copy(idx_hbm.at[b, pl.ds(n0, TPW)], raw_v)
        def deint(g, _):
            for q in range(NUM_Q):
                v = plsc.load_gather(raw_v, [iota + g * L, jnp.full((L,), q, jnp.int32)])
                idxq[q, pl.ds(g * L, L)] = v
            return _

        lax.fori_loop(0, TPW // L, deint, 0, unroll=False)

    with jax.named_scope("ph_zero"):
        zeros = jnp.zeros((L,), jnp.float32)

        def zrow(t, _):
            for r in range(CODE_DIM // L):
                acc[t, pl.ds(r * L, L)] = zeros
            return _

        lax.fori_loop(0, TPW, zrow, 0, unroll=4)

    with jax.named_scope("ph_main"):
        copies = []
        for c in range(NCHUNK):
            dst = acc.at[pl.ds(c * CHUNK_T, CHUNK_T)]
            for q in range(NUM_Q):
                src = cb_hbm.at[q].at[idxq.at[q, pl.ds(c * CHUNK_T, CHUNK_T)]]
                copies.append(pltpu.make_async_copy(src, dst, gsems[c]))
        for cp in copies:
            cp.start(add=True)

    with jax.named_scope("ph_tpose"):
        def prow(t, _):
            for r in range(CODE_DIM // L):
                acc65[pl.ds(t * ROWP + r * L, L)] = acc[t, pl.ds(r * L, L)]
            return _

        colbase = iota * ROWP

        def tpass(g, _):
            rowb = g * L
            for d in range(CODE_DIM):
                v = plsc.load_gather(acc65, [colbase + (rowb * ROWP + d)])
                accT[pl.ds(d * TPW + rowb, L)] = v
            return _

        for c in range(NCHUNK):
            for cp in copies[c * NUM_Q : (c + 1) * NUM_Q]:
                cp.wait()
            lax.fori_loop(c * CHUNK_T, (c + 1) * CHUNK_T, prow, 0, unroll=4)
            lax.fori_loop(c * GPC, (c + 1) * GPC, tpass, 0, unroll=False)

    with jax.named_scope("ph_out"):
        def ocopy(d):
            return pltpu.make_async_copy(
                accT.at[pl.ds(d * TPW, TPW)],
                out_hbm.at[b, d, pl.ds(n0, TPW)],
                osem,
            )

        def ostart(d, _):
            ocopy(d).start()
            return _

        def odrain(d, _):
            ocopy(d).wait()
            return _

        lax.fori_loop(0, CODE_DIM, ostart, 0, unroll=False)
        lax.fori_loop(0, CODE_DIM, odrain, 0, unroll=False)


@jax.jit
def _vq_lookup(indices, codebooks):
    mesh = plsc.VectorSubcoreMesh(
        core_axis_name="c", subcore_axis_name="s", num_cores=NC, num_subcores=NS
    )
    f = pl.kernel(
        _body,
        out_type=jax.ShapeDtypeStruct((B, CODE_DIM, N), jnp.float32),
        mesh=mesh,
        compiler_params=pltpu.CompilerParams(
            use_tc_tiling_on_sc=False, needs_layout_passes=False
        ),
        scratch_types=[
            pltpu.VMEM((TPW, NUM_Q), jnp.int32),
            pltpu.VMEM((NUM_Q, TPW), jnp.int32),
            pltpu.VMEM((TPW, CODE_DIM), jnp.float32),
            pltpu.VMEM((TPW * ROWP,), jnp.float32),
            pltpu.VMEM((CODE_DIM * TPW,), jnp.float32),
        ]
        + [pltpu.SemaphoreType.DMA] * (NCHUNK + 1),
    )
    return f(indices, codebooks)


def kernel(indices, codebooks):
    if indices.dtype != jnp.int32:
        indices = indices.astype(jnp.int32)
    return _vq_lookup(indices, codebooks)


# speedup vs baseline: 1.4341x; 1.0418x over previous
import jax
import jax.numpy as jnp
from jax import lax
from jax.experimental import pallas as pl
from jax.experimental.pallas import tpu as pltpu
from jax.experimental.pallas import tpu_sc as plsc

NUM_Q = 8
CODEBOOK_SIZE = 1024
CODE_DIM = 64
B, N = 16, 576

NC, NS, L = 2, 16, 16
NW = NC * NS
T = B * N
TPW = T // NW
CHUNK_T = 48
NCHUNK = TPW // CHUNK_T
ROWP = 65
GPC = TPW // L // NCHUNK


def _body(idx_hbm, cb_hbm, out_hbm, raw_v, idxq, acc, acc65, accT, *sems):
    gsems, osem = sems[:NCHUNK], sems[NCHUNK]
    wid = lax.axis_index("c") * NS + lax.axis_index("s")
    b = wid // 2
    n0 = (wid % 2) * TPW

    iota = lax.iota(jnp.int32, L)
    zeros = jnp.zeros((L,), jnp.float32)

    with jax.named_scope("ph_idx"):
        pltpu.sync_copy(idx_hbm.at[pl.ds(wid * TPW * NUM_Q, TPW * NUM_Q)], raw_v)

    copies = []
    with jax.named_scope("ph_main"):
        for c in range(NCHUNK):
            def deint(g, _):
                for q in range(NUM_Q):
                    v = plsc.load_gather(raw_v, [iota * NUM_Q + (g * L * NUM_Q + q)])
                    idxq[q, pl.ds(g * L, L)] = v
                return _

            def zrow(t, _):
                for r in range(CODE_DIM // L):
                    acc[t, pl.ds(r * L, L)] = zeros
                return _

            lax.fori_loop(c * GPC, (c + 1) * GPC, deint, 0, unroll=True)
            lax.fori_loop(c * CHUNK_T, (c + 1) * CHUNK_T, zrow, 0, unroll=8)
            dst = acc.at[pl.ds(c * CHUNK_T, CHUNK_T)]
            for q in range(NUM_Q):
                src = cb_hbm.at[q].at[idxq.at[q, pl.ds(c * CHUNK_T, CHUNK_T)]]
                cp = pltpu.make_async_copy(src, dst, gsems[c])
                cp.start(add=True)
                copies.append(cp)

    with jax.named_scope("ph_tpose"):
        def prow(t, _):
            for r in range(CODE_DIM // L):
                acc65[pl.ds(t * ROWP + r * L, L)] = acc[t, pl.ds(r * L, L)]
            return _

        colbase = iota * ROWP

        def tpass(g, _):
            rowb = g * L
            for d in range(CODE_DIM):
                v = plsc.load_gather(acc65, [colbase + (rowb * ROWP + d)])
                accT[pl.ds(d * TPW + rowb, L)] = v
            return _

        for c in range(NCHUNK):
            for cp in copies[c * NUM_Q : (c + 1) * NUM_Q]:
                cp.wait()
            lax.fori_loop(c * CHUNK_T, (c + 1) * CHUNK_T, prow, 0, unroll=8)
            lax.fori_loop(c * GPC, (c + 1) * GPC, tpass, 0, unroll=False)

    with jax.named_scope("ph_out"):
        def ocopy(d):
            return pltpu.make_async_copy(
                accT.at[pl.ds(d * TPW, TPW)],
                out_hbm.at[b, d, pl.ds(n0, TPW)],
                osem,
            )

        def ostart(d, _):
            ocopy(d).start()
            return _

        def odrain(d, _):
            ocopy(d).wait()
            return _

        lax.fori_loop(0, CODE_DIM, ostart, 0, unroll=False)
        lax.fori_loop(0, CODE_DIM, odrain, 0, unroll=False)


@jax.jit
def _vq_lookup(indices, codebooks):
    mesh = plsc.VectorSubcoreMesh(
        core_axis_name="c", subcore_axis_name="s", num_cores=NC, num_subcores=NS
    )
    f = pl.kernel(
        _body,
        out_type=jax.ShapeDtypeStruct((B, CODE_DIM, N), jnp.float32),
        mesh=mesh,
        compiler_params=pltpu.CompilerParams(
            use_tc_tiling_on_sc=False, needs_layout_passes=False
        ),
        scratch_types=[
            pltpu.VMEM((TPW * NUM_Q,), jnp.int32),
            pltpu.VMEM((NUM_Q, TPW), jnp.int32),
            pltpu.VMEM((TPW, CODE_DIM), jnp.float32),
            pltpu.VMEM((TPW * ROWP,), jnp.float32),
            pltpu.VMEM((CODE_DIM * TPW,), jnp.float32),
        ]
        + [pltpu.SemaphoreType.DMA] * (NCHUNK + 1),
    )
    return f(indices, codebooks)


def kernel(indices, codebooks):
    if indices.dtype != jnp.int32:
        indices = indices.astype(jnp.int32)
    return _vq_lookup(indices.reshape(T * NUM_Q), codebooks)
